# initial kernel scaffold (unmeasured)
import jax
import jax.numpy as jnp
from jax import lax
from jax.experimental import pallas as pl
from jax.experimental.pallas import tpu as pltpu

B, H, D, BS = 8, 8, 128, 16
NPAGES = 512
NSLOTS = 512
C = 32
KEYS = C * BS
NCHUNK_MAX = NSLOTS // C
NEG = -1e30
SCALE = D ** -0.5


def _body(bt_ref, len_ref, mask_ref, q_ref, k_hbm, v_hbm, out_ref,
          kbuf, vbuf, cm_send, cm_recv, ksem, vsem, sem_s, sem_r):
    my_x = lax.axis_index("x")
    my_y = lax.axis_index("y")
    peer = (1 - my_x, my_y)

    bar = pltpu.get_barrier_semaphore()
    pl.semaphore_signal(bar, inc=1, device_id=peer,
                        device_id_type=pl.DeviceIdType.MESH)
    pl.semaphore_wait(bar, 1)

    def start_chunk(i, jc, slot):
        base = jc * C
        for c in range(C):
            p = bt_ref[i, base + c]
            pltpu.make_async_copy(k_hbm.at[p], kbuf.at[slot, c],
                                  ksem.at[slot]).start()
            pltpu.make_async_copy(v_hbm.at[p], vbuf.at[slot, c],
                                  vsem.at[slot]).start()

    def wait_chunk(slot):
        for c in range(C):
            pltpu.make_async_copy(k_hbm.at[0], kbuf.at[slot, c],
                                  ksem.at[slot]).wait()
            pltpu.make_async_copy(v_hbm.at[0], vbuf.at[slot, c],
                                  vsem.at[slot]).wait()

    for i in range(B):
        len_i = len_ref[i]
        n_chunks = (len_i + (C - 1)) // C
        q_i = q_ref[i, 0] * SCALE
        start_chunk(i, 0, 0)

        def chunk_body(jc, carry, i=i, n_chunks=n_chunks, q_i=q_i):
            m, l, o = carry
            slot = lax.rem(jc, 2)

            @pl.when(jc + 1 < n_chunks)
            def _():
                start_chunk(i, jc + 1, lax.rem(jc + 1, 2))

            wait_chunk(slot)
            kc = kbuf[slot].reshape(KEYS, H, D)
            vc = vbuf[slot].reshape(KEYS, H, D)
            mask = mask_ref[pl.ds(i, 1), pl.ds(jc * KEYS, KEYS)]
            s = lax.dot_general(
                q_i, kc, (((1,), (2,)), ((0,), (1,))),
                preferred_element_type=jnp.float32)
            s = jnp.where(mask > 0, s, NEG)
            m_new = jnp.maximum(m, jnp.max(s, axis=1, keepdims=True))
            alpha = jnp.exp(m - m_new)
            p = jnp.exp(s - m_new) * mask
            l_new = l * alpha + jnp.sum(p, axis=1, keepdims=True)
            o_new = o * alpha + lax.dot_general(
                p, vc, (((1,), (0,)), ((0,), (1,))),
                preferred_element_type=jnp.float32)
            return m_new, l_new, o_new

        m0 = jnp.full((H, 1), NEG, jnp.float32)
        l0 = jnp.zeros((H, 1), jnp.float32)
        o0 = jnp.zeros((H, D), jnp.float32)
        m, l, o = lax.fori_loop(0, n_chunks, chunk_body, (m0, l0, o0))

        cm_send[i, :, 0:D] = o
        cm_send[i, :, D:2 * D] = jnp.broadcast_to(m, (H, D))
        cm_send[i, :, 2 * D:3 * D] = jnp.broadcast_to(l, (H, D))

    rdma = pltpu.make_async_remote_copy(
        src_ref=cm_send, dst_ref=cm_recv,
        send_sem=sem_s, recv_sem=sem_r,
        device_id=peer, device_id_type=pl.DeviceIdType.MESH)
    rdma.start()
    rdma.wait()

    s_l = cm_send[...]
    s_r = cm_recv[...]
    o_l, m_l, l_l = s_l[:, :, :D], s_l[:, :, D:2 * D], s_l[:, :, 2 * D:]
    o_r, m_r, l_r = s_r[:, :, :D], s_r[:, :, D:2 * D], s_r[:, :, 2 * D:]
    m_t = jnp.maximum(m_l, m_r)
    a_l = jnp.exp(m_l - m_t)
    a_r = jnp.exp(m_r - m_t)
    l_t = l_l * a_l + l_r * a_r
    out_ref[:, 0, :, :] = (o_l * a_l + o_r * a_r) / l_t


def kernel(Q, K, V, bt, lens):
    my_x = lax.axis_index("x")

    slot_idx = jnp.arange(NSLOTS)[None, :]
    valid = (slot_idx < lens[:, None]) & ((bt // NPAGES) == my_x)
    mask = jnp.broadcast_to(valid[:, :, None], (B, NSLOTS, BS))
    mask = mask.reshape(B, NSLOTS * BS).astype(jnp.float32)
    bt_local = jnp.clip(bt - my_x * NPAGES, 0, NPAGES - 1).astype(jnp.int32)

    return pl.pallas_call(
        _body,
        out_shape=jax.ShapeDtypeStruct((B, 1, H, D), jnp.float32),
        in_specs=[
            pl.BlockSpec(memory_space=pltpu.SMEM),
            pl.BlockSpec(memory_space=pltpu.SMEM),
            pl.BlockSpec(memory_space=pltpu.VMEM),
            pl.BlockSpec(memory_space=pltpu.VMEM),
            pl.BlockSpec(memory_space=pltpu.ANY),
            pl.BlockSpec(memory_space=pltpu.ANY),
        ],
        out_specs=pl.BlockSpec(memory_space=pltpu.VMEM),
        scratch_shapes=[
            pltpu.VMEM((2, C, BS, H, D), jnp.float32),
            pltpu.VMEM((2, C, BS, H, D), jnp.float32),
            pltpu.VMEM((B, H, 3 * D), jnp.float32),
            pltpu.VMEM((B, H, 3 * D), jnp.float32),
            pltpu.SemaphoreType.DMA((2,)),
            pltpu.SemaphoreType.DMA((2,)),
            pltpu.SemaphoreType.DMA,
            pltpu.SemaphoreType.DMA,
        ],
        compiler_params=pltpu.CompilerParams(collective_id=0),
    )(bt_local, lens.astype(jnp.int32), mask, Q, K, V)


# baseline (device time: 1065814 ns/iter reference)
import jax
import jax.numpy as jnp
from jax import lax
from jax.experimental import pallas as pl
from jax.experimental.pallas import tpu as pltpu

B, H, D, BS = 8, 8, 128, 16
NPAGES = 512
NSLOTS = 512
C = 32
KEYS = C * BS
NCHUNK_MAX = NSLOTS // C
NEG = -1e30
SCALE = D ** -0.5


def _body(bt_ref, len_ref, mask_ref, q_ref, k_hbm, v_hbm, out_ref,
          kbuf, vbuf, cm_send, cm_recv, ksem, vsem, sem_s, sem_r):
    my_x = lax.axis_index("x")
    my_y = lax.axis_index("y")
    peer = (1 - my_x, my_y)

    bar = pltpu.get_barrier_semaphore()
    pl.semaphore_signal(bar, inc=1, device_id=peer,
                        device_id_type=pl.DeviceIdType.MESH)
    pl.semaphore_wait(bar, 1)

    def start_chunk(i, jc, slot):
        base = jc * C
        for c in range(C):
            p = bt_ref[i, base + c]
            pltpu.make_async_copy(k_hbm.at[p], kbuf.at[slot, c],
                                  ksem.at[slot]).start()
            pltpu.make_async_copy(v_hbm.at[p], vbuf.at[slot, c],
                                  vsem.at[slot]).start()

    def wait_chunk(slot):
        for c in range(C):
            pltpu.make_async_copy(k_hbm.at[0], kbuf.at[slot, c],
                                  ksem.at[slot]).wait()
            pltpu.make_async_copy(v_hbm.at[0], vbuf.at[slot, c],
                                  vsem.at[slot]).wait()

    for i in range(B):
        len_i = len_ref[i]
        n_chunks = (len_i + (C - 1)) // C
        q_i = q_ref[i, 0] * SCALE
        start_chunk(i, 0, 0)

        def chunk_body(jc, carry, i=i, n_chunks=n_chunks, q_i=q_i):
            m, l, o = carry
            slot = lax.rem(jc, 2)

            @pl.when(jc + 1 < n_chunks)
            def _():
                start_chunk(i, jc + 1, lax.rem(jc + 1, 2))

            wait_chunk(slot)
            kc = kbuf[slot].reshape(KEYS, H, D)
            vc = vbuf[slot].reshape(KEYS, H, D)
            mask = mask_ref[pl.ds(i, 1), pl.ds(jc * KEYS, KEYS)]
            s = lax.dot_general(
                q_i, kc, (((1,), (2,)), ((0,), (1,))),
                preferred_element_type=jnp.float32)
            s = jnp.where(mask > 0, s, NEG)
            m_new = jnp.maximum(m, jnp.max(s, axis=1, keepdims=True))
            alpha = jnp.exp(m - m_new)
            p = jnp.exp(s - m_new) * mask
            l_new = l * alpha + jnp.sum(p, axis=1, keepdims=True)
            o_new = o * alpha + lax.dot_general(
                p, vc, (((1,), (0,)), ((0,), (1,))),
                preferred_element_type=jnp.float32)
            return m_new, l_new, o_new

        m0 = jnp.full((H, 1), NEG, jnp.float32)
        l0 = jnp.zeros((H, 1), jnp.float32)
        o0 = jnp.zeros((H, D), jnp.float32)
        m, l, o = lax.fori_loop(0, n_chunks, chunk_body, (m0, l0, o0))

        cm_send[i, :, 0:D] = o
        cm_send[i, :, D:2 * D] = jnp.broadcast_to(m, (H, D))
        cm_send[i, :, 2 * D:3 * D] = jnp.broadcast_to(l, (H, D))

    rdma = pltpu.make_async_remote_copy(
        src_ref=cm_send, dst_ref=cm_recv,
        send_sem=sem_s, recv_sem=sem_r,
        device_id=peer, device_id_type=pl.DeviceIdType.MESH)
    rdma.start()
    rdma.wait()

    s_l = cm_send[...]
    s_r = cm_recv[...]
    o_l, m_l, l_l = s_l[:, :, :D], s_l[:, :, D:2 * D], s_l[:, :, 2 * D:]
    o_r, m_r, l_r = s_r[:, :, :D], s_r[:, :, D:2 * D], s_r[:, :, 2 * D:]
    m_t = jnp.maximum(m_l, m_r)
    a_l = jnp.exp(m_l - m_t)
    a_r = jnp.exp(m_r - m_t)
    l_t = l_l * a_l + l_r * a_r
    out_ref[:, 0, :, :] = (o_l * a_l + o_r * a_r) / l_t


def kernel(Q, K, V, bt, lens):
    my_x = lax.axis_index("x")

    slot_idx = jnp.arange(NSLOTS)[None, :]
    valid = (slot_idx < lens[:, None]) & ((bt // NPAGES) == my_x)
    mask = jnp.broadcast_to(valid[:, :, None], (B, NSLOTS, BS))
    mask = mask.reshape(B, NSLOTS * BS).astype(jnp.float32)
    bt_local = jnp.clip(bt - my_x * NPAGES, 0, NPAGES - 1).astype(jnp.int32)

    return pl.pallas_call(
        _body,
        out_shape=jax.ShapeDtypeStruct((B, 1, H, D), jnp.float32),
        in_specs=[
            pl.BlockSpec(memory_space=pltpu.SMEM),
            pl.BlockSpec(memory_space=pltpu.SMEM),
            pl.BlockSpec(memory_space=pltpu.VMEM),
            pl.BlockSpec(memory_space=pltpu.VMEM),
            pl.BlockSpec(memory_space=pl.ANY),
            pl.BlockSpec(memory_space=pl.ANY),
        ],
        out_specs=pl.BlockSpec(memory_space=pltpu.VMEM),
        scratch_shapes=[
            pltpu.VMEM((2, C, BS, H, D), jnp.float32),
            pltpu.VMEM((2, C, BS, H, D), jnp.float32),
            pltpu.VMEM((B, H, 3 * D), jnp.float32),
            pltpu.VMEM((B, H, 3 * D), jnp.float32),
            pltpu.SemaphoreType.DMA((2,)),
            pltpu.SemaphoreType.DMA((2,)),
            pltpu.SemaphoreType.DMA,
            pltpu.SemaphoreType.DMA,
        ],
        compiler_params=pltpu.CompilerParams(
            collective_id=0, vmem_limit_bytes=100 * 1024 * 1024),
    )(bt_local, lens.astype(jnp.int32), mask, Q, K, V)


# device time: 247169 ns/iter; 4.3121x vs baseline; 4.3121x over previous
import jax
import jax.numpy as jnp
from jax import lax
from jax.experimental import pallas as pl
from jax.experimental.pallas import tpu as pltpu

B, H, D, BS = 8, 8, 128, 16
HD = H * D
NPAGES = 512
NSLOTS = 512
C = 32
KEYS = C * BS
NCHUNK_MAX = NSLOTS // C
NEG = -1e30
BIG = 1e30
SCALE = D ** -0.5


def _body(bt_ref, len_ref, valid_ref, qblk_ref, k_hbm, v_hbm, out_ref,
          kbuf, vbuf, cm_send, cm_recv, ksem, vsem, sem_s, sem_r):
    my_x = lax.axis_index("x")
    my_y = lax.axis_index("y")
    peer = (1 - my_x, my_y)

    bar = pltpu.get_barrier_semaphore()
    pl.semaphore_signal(bar, inc=1, device_id=peer,
                        device_id_type=pl.DeviceIdType.MESH)
    pl.semaphore_wait(bar, 1)

    row = lax.broadcasted_iota(jnp.int32, (H, HD), 0)
    col = lax.broadcasted_iota(jnp.int32, (H, HD), 1)
    R = (col // D == row).astype(jnp.float32)
    pidx = lax.broadcasted_iota(jnp.int32, (KEYS, 1), 0) // BS

    def start_chunk(i, jc, slot):
        base = jc * C
        for c in range(C):
            p = bt_ref[i, base + c]
            pltpu.make_async_copy(k_hbm.at[p], kbuf.at[slot, c],
                                  ksem.at[slot]).start()
            pltpu.make_async_copy(v_hbm.at[p], vbuf.at[slot, c],
                                  vsem.at[slot]).start()

    def wait_chunk(slot):
        for c in range(C):
            pltpu.make_async_copy(k_hbm.at[0], kbuf.at[slot, c],
                                  ksem.at[slot]).wait()
            pltpu.make_async_copy(v_hbm.at[0], vbuf.at[slot, c],
                                  vsem.at[slot]).wait()

    for i in range(B):
        len_i = len_ref[i]
        n_chunks = (len_i + (C - 1)) // C
        qb = qblk_ref[i]
        start_chunk(i, 0, 0)

        def chunk_body(jc, carry, i=i, n_chunks=n_chunks, qb=qb):
            m, l, o = carry
            slot = lax.rem(jc, 2)

            @pl.when(jc + 1 < n_chunks)
            def _():
                start_chunk(i, jc + 1, lax.rem(jc + 1, 2))

            wait_chunk(slot)
            kmat = kbuf[slot].reshape(KEYS, HD)
            vmat = vbuf[slot].reshape(KEYS, HD)
            vcol = jnp.zeros((KEYS, 1), jnp.float32)
            for c in range(C):
                vb = valid_ref[i, jc * C + c].astype(jnp.float32)
                vcol = jnp.where(pidx == c, vb, vcol)
            s = lax.dot_general(
                kmat, qb, (((1,), (0,)), ((), ())),
                preferred_element_type=jnp.float32)
            s = s + (vcol - 1.0) * BIG
            m_new = jnp.maximum(m, jnp.max(s, axis=0, keepdims=True))
            alpha = jnp.exp(m - m_new)
            p = jnp.exp(s - m_new) * vcol
            l_new = l * alpha + jnp.sum(p, axis=0, keepdims=True)
            pexp = lax.dot_general(
                p, R, (((1,), (0,)), ((), ())),
                preferred_element_type=jnp.float32)
            alpha_f = lax.dot_general(
                alpha, R, (((1,), (0,)), ((), ())),
                preferred_element_type=jnp.float32)
            o_new = o * alpha_f + jnp.sum(pexp * vmat, axis=0, keepdims=True)
            return m_new, l_new, o_new

        m0 = jnp.full((1, H), NEG, jnp.float32)
        l0 = jnp.zeros((1, H), jnp.float32)
        o0 = jnp.zeros((1, HD), jnp.float32)
        m, l, o = lax.fori_loop(0, n_chunks, chunk_body, (m0, l0, o0))

        cm_send[pl.ds(i, 1), 0:HD] = o
        cm_send[pl.ds(i, 1), HD:HD + H] = m
        cm_send[pl.ds(i, 1), HD + H:HD + 2 * H] = l

    rdma = pltpu.make_async_remote_copy(
        src_ref=cm_send, dst_ref=cm_recv,
        send_sem=sem_s, recv_sem=sem_r,
        device_id=peer, device_id_type=pl.DeviceIdType.MESH)
    rdma.start()
    rdma.wait()

    s_l = cm_send[...]
    s_r = cm_recv[...]
    o_l, m_l, l_l = s_l[:, :HD], s_l[:, HD:HD + H], s_l[:, HD + H:]
    o_r, m_r, l_r = s_r[:, :HD], s_r[:, HD:HD + H], s_r[:, HD + H:]
    m_t = jnp.maximum(m_l, m_r)
    a_l = jnp.exp(m_l - m_t)
    a_r = jnp.exp(m_r - m_t)
    l_t = l_l * a_l + l_r * a_r
    ex = lambda x: lax.dot_general(
        x, R, (((1,), (0,)), ((), ())),
        preferred_element_type=jnp.float32)
    out_ref[...] = (o_l * ex(a_l) + o_r * ex(a_r)) / ex(l_t)


def kernel(Q, K, V, bt, lens):
    my_x = lax.axis_index("x")

    slot_idx = jnp.arange(NSLOTS)[None, :]
    valid = ((slot_idx < lens[:, None]) & ((bt // NPAGES) == my_x)
             ).astype(jnp.int32)

    bt_local = jnp.clip(bt - my_x * NPAGES, 0, NPAGES - 1).astype(jnp.int32)

    q = Q[:, 0] * SCALE
    eye = jnp.eye(H, dtype=jnp.float32)
    qblk = (q[:, :, :, None] * eye[:, None, :][None]).reshape(B, HD, H)

    K2 = K.reshape(NPAGES, BS, HD)
    V2 = V.reshape(NPAGES, BS, HD)

    out = pl.pallas_call(
        _body,
        out_shape=jax.ShapeDtypeStruct((B, HD), jnp.float32),
        in_specs=[
            pl.BlockSpec(memory_space=pltpu.SMEM),
            pl.BlockSpec(memory_space=pltpu.SMEM),
            pl.BlockSpec(memory_space=pltpu.SMEM),
            pl.BlockSpec(memory_space=pltpu.VMEM),
            pl.BlockSpec(memory_space=pl.ANY),
            pl.BlockSpec(memory_space=pl.ANY),
        ],
        out_specs=pl.BlockSpec(memory_space=pltpu.VMEM),
        scratch_shapes=[
            pltpu.VMEM((2, C, BS, HD), jnp.float32),
            pltpu.VMEM((2, C, BS, HD), jnp.float32),
            pltpu.VMEM((B, HD + 2 * H), jnp.float32),
            pltpu.VMEM((B, HD + 2 * H), jnp.float32),
            pltpu.SemaphoreType.DMA((2,)),
            pltpu.SemaphoreType.DMA((2,)),
            pltpu.SemaphoreType.DMA,
            pltpu.SemaphoreType.DMA,
        ],
        compiler_params=pltpu.CompilerParams(
            collective_id=0, vmem_limit_bytes=100 * 1024 * 1024),
    )(bt_local, lens.astype(jnp.int32), valid, qblk, K2, V2)
    return out.reshape(B, 1, H, D)


# device time: 206149 ns/iter; 5.1701x vs baseline; 1.1990x over previous
import jax
import jax.numpy as jnp
from jax import lax
from jax.experimental import pallas as pl
from jax.experimental.pallas import tpu as pltpu

B, H, D, BS = 8, 8, 128, 16
BL = 4
HD = H * D
NPAGES = 512
NSLOTS = 512
C = 32
KEYS = C * BS
NEG = -1e30
BIG = 2e30
SCALE = D ** -0.5


def _body(bt_ref, len_ref, valid_ref, qblk_ref, k_hbm, v_hbm, out_ref,
          kbuf, vbuf, cm_send, cm_recv, yfin, yrecv,
          ksem, vsem, sem_xs, sem_xr, sem_ys, sem_yr):
    my_x = lax.axis_index("x")
    my_y = lax.axis_index("y")
    xpeer = (1 - my_x, my_y)
    ypeer = (my_x, 1 - my_y)

    bar = pltpu.get_barrier_semaphore()
    pl.semaphore_signal(bar, inc=1, device_id=xpeer,
                        device_id_type=pl.DeviceIdType.MESH)
    pl.semaphore_signal(bar, inc=1, device_id=ypeer,
                        device_id_type=pl.DeviceIdType.MESH)
    pl.semaphore_wait(bar, 2)

    row = lax.broadcasted_iota(jnp.int32, (H, HD), 0)
    col = lax.broadcasted_iota(jnp.int32, (H, HD), 1)
    R = (col // D == row).astype(jnp.float32)
    pidx = lax.broadcasted_iota(jnp.int32, (KEYS, 1), 0) // BS

    def start_chunk(i, jc, slot):
        base = jc * C
        for c in range(C):
            p = bt_ref[i, base + c]
            pltpu.make_async_copy(k_hbm.at[p], kbuf.at[slot, c],
                                  ksem.at[slot]).start()
            pltpu.make_async_copy(v_hbm.at[p], vbuf.at[slot, c],
                                  vsem.at[slot]).start()

    def wait_chunk(slot):
        for c in range(C):
            pltpu.make_async_copy(k_hbm.at[0], kbuf.at[slot, c],
                                  ksem.at[slot]).wait()
            pltpu.make_async_copy(v_hbm.at[0], vbuf.at[slot, c],
                                  vsem.at[slot]).wait()

    for i in range(BL):
        len_i = len_ref[i]
        n_chunks = (len_i + (C - 1)) // C
        qb = qblk_ref[i]
        start_chunk(i, 0, 0)

        def chunk_body(jc, carry, i=i, n_chunks=n_chunks, qb=qb):
            m, l, o = carry
            slot = lax.rem(jc, 2)

            @pl.when(jc + 1 < n_chunks)
            def _():
                start_chunk(i, jc + 1, lax.rem(jc + 1, 2))

            wait_chunk(slot)
            kmat = kbuf[slot].reshape(KEYS, HD)
            vmat = vbuf[slot].reshape(KEYS, HD)
            vcol = jnp.zeros((KEYS, 1), jnp.float32)
            for c in range(C):
                vb = valid_ref[i, jc * C + c].astype(jnp.float32)
                vcol = jnp.where(pidx == c, vb, vcol)
            s = lax.dot_general(kmat, qb, (((1,), (0,)), ((), ())),
                                preferred_element_type=jnp.float32)
            s = s + (vcol - 1.0) * BIG
            m_new = jnp.maximum(m, jnp.max(s, axis=0, keepdims=True))
            alpha = jnp.exp(m - m_new)
            p = jnp.exp(s - m_new) * vcol
            l_new = l * alpha + jnp.sum(p, axis=0, keepdims=True)
            pexp = lax.dot_general(p, R, (((1,), (0,)), ((), ())),
                                   preferred_element_type=jnp.float32)
            alpha_f = lax.dot_general(alpha, R, (((1,), (0,)), ((), ())),
                                      preferred_element_type=jnp.float32)
            o_new = o * alpha_f + jnp.sum(pexp * vmat, axis=0,
                                          keepdims=True)
            return m_new, l_new, o_new

        m0 = jnp.full((1, H), NEG, jnp.float32)
        l0 = jnp.zeros((1, H), jnp.float32)
        o0 = jnp.zeros((1, HD), jnp.float32)
        m, l, o = lax.fori_loop(0, n_chunks, chunk_body, (m0, l0, o0))

        cm_send[pl.ds(i, 1), 0:HD] = o
        cm_send[pl.ds(i, 1), HD:HD + H] = m
        cm_send[pl.ds(i, 1), HD + H:HD + 2 * H] = l

    rdma_x = pltpu.make_async_remote_copy(
        src_ref=cm_send, dst_ref=cm_recv,
        send_sem=sem_xs, recv_sem=sem_xr,
        device_id=xpeer, device_id_type=pl.DeviceIdType.MESH)
    rdma_x.start()
    rdma_x.wait()

    s_l = cm_send[...]
    s_r = cm_recv[...]
    o_l, m_l, l_l = s_l[:, :HD], s_l[:, HD:HD + H], s_l[:, HD + H:]
    o_r, m_r, l_r = s_r[:, :HD], s_r[:, HD:HD + H], s_r[:, HD + H:]
    m_t = jnp.maximum(m_l, m_r)
    a_l = jnp.exp(m_l - m_t)
    a_r = jnp.exp(m_r - m_t)
    l_t = l_l * a_l + l_r * a_r
    ex = lambda x: lax.dot_general(
        x, R, (((1,), (0,)), ((), ())),
        preferred_element_type=jnp.float32)
    yfin[...] = (o_l * ex(a_l) + o_r * ex(a_r)) / ex(l_t)

    rdma_y = pltpu.make_async_remote_copy(
        src_ref=yfin, dst_ref=yrecv,
        send_sem=sem_ys, recv_sem=sem_yr,
        device_id=ypeer, device_id_type=pl.DeviceIdType.MESH)
    rdma_y.start()
    rdma_y.wait()

    @pl.when(my_y == 0)
    def _():
        out_ref[0:BL, :] = yfin[...]
        out_ref[BL:B, :] = yrecv[...]

    @pl.when(my_y == 1)
    def _():
        out_ref[BL:B, :] = yfin[...]
        out_ref[0:BL, :] = yrecv[...]


def kernel(Q, K, V, bt, lens):
    my_x = lax.axis_index("x")
    my_y = lax.axis_index("y")

    b0 = my_y * BL
    Qh = lax.dynamic_slice_in_dim(Q, b0, BL, axis=0)
    bth = lax.dynamic_slice_in_dim(bt, b0, BL, axis=0)
    lensh = lax.dynamic_slice_in_dim(lens, b0, BL, axis=0)

    slot_idx = jnp.arange(NSLOTS)[None, :]
    valid = ((slot_idx < lensh[:, None]) & ((bth // NPAGES) == my_x)
             ).astype(jnp.int32)
    bt_local = jnp.clip(bth - my_x * NPAGES, 0, NPAGES - 1).astype(jnp.int32)

    q = Qh[:, 0] * SCALE
    eye = jnp.eye(H, dtype=jnp.float32)
    qblk = (q[:, :, :, None] * eye[:, None, :][None]).reshape(BL, HD, H)

    K2 = K.reshape(NPAGES, BS, HD)
    V2 = V.reshape(NPAGES, BS, HD)

    out = pl.pallas_call(
        _body,
        out_shape=jax.ShapeDtypeStruct((B, HD), jnp.float32),
        in_specs=[
            pl.BlockSpec(memory_space=pltpu.SMEM),
            pl.BlockSpec(memory_space=pltpu.SMEM),
            pl.BlockSpec(memory_space=pltpu.SMEM),
            pl.BlockSpec(memory_space=pltpu.VMEM),
            pl.BlockSpec(memory_space=pl.ANY),
            pl.BlockSpec(memory_space=pl.ANY),
        ],
        out_specs=pl.BlockSpec(memory_space=pltpu.VMEM),
        scratch_shapes=[
            pltpu.VMEM((2, C, BS, HD), jnp.float32),
            pltpu.VMEM((2, C, BS, HD), jnp.float32),
            pltpu.VMEM((BL, HD + 2 * H), jnp.float32),
            pltpu.VMEM((BL, HD + 2 * H), jnp.float32),
            pltpu.VMEM((BL, HD), jnp.float32),
            pltpu.VMEM((BL, HD), jnp.float32),
            pltpu.SemaphoreType.DMA((2,)),
            pltpu.SemaphoreType.DMA((2,)),
            pltpu.SemaphoreType.DMA,
            pltpu.SemaphoreType.DMA,
            pltpu.SemaphoreType.DMA,
            pltpu.SemaphoreType.DMA,
        ],
        compiler_params=pltpu.CompilerParams(
            collective_id=0, vmem_limit_bytes=100 * 1024 * 1024),
    )(bt_local, lensh.astype(jnp.int32), valid, qblk, K2, V2)
    return out.reshape(B, 1, H, D)


# device time: 121854 ns/iter; 8.7466x vs baseline; 1.6918x over previous
import jax
import jax.numpy as jnp
from jax import lax
from jax.experimental import pallas as pl
from jax.experimental.pallas import tpu as pltpu

B, H, D, BS = 8, 8, 128, 16
BL = 4
BLH = BL * H
HD = H * D
NPAGES = 512
NSLOTS = 512
C = 32
KEYS = C * BS
NCH = NPAGES // C
NEG = -1e30
BIG = 2e30
SCALE = D ** -0.5


def _body(cntT_ref, qb_ref, k_hbm, v_hbm, out_ref,
          kbuf, vbuf, cm_send, cm_recv, yfin, yrecv,
          ksem, vsem, sem_xs, sem_xr, sem_ys, sem_yr):
    my_x = lax.axis_index("x")
    my_y = lax.axis_index("y")
    xpeer = (1 - my_x, my_y)
    ypeer = (my_x, 1 - my_y)

    bar = pltpu.get_barrier_semaphore()
    pl.semaphore_signal(bar, inc=1, device_id=xpeer,
                        device_id_type=pl.DeviceIdType.MESH)
    pl.semaphore_signal(bar, inc=1, device_id=ypeer,
                        device_id_type=pl.DeviceIdType.MESH)
    pl.semaphore_wait(bar, 2)

    row = lax.broadcasted_iota(jnp.int32, (H, HD), 0)
    col = lax.broadcasted_iota(jnp.int32, (H, HD), 1)
    R = (col // D == row).astype(jnp.float32)
    r4r = lax.broadcasted_iota(jnp.int32, (BL, BLH), 0)
    r4c = lax.broadcasted_iota(jnp.int32, (BL, BLH), 1)
    R4 = (r4c // H == r4r).astype(jnp.float32)
    kidx = lax.broadcasted_iota(jnp.int32, (KEYS, C), 0) // BS
    cidx = lax.broadcasted_iota(jnp.int32, (KEYS, C), 1)
    Rp = (kidx == cidx).astype(jnp.float32)

    def start_chunk(jc, slot):
        pltpu.make_async_copy(k_hbm.at[pl.ds(jc * C, C)], kbuf.at[slot],
                              ksem.at[slot]).start()
        pltpu.make_async_copy(v_hbm.at[pl.ds(jc * C, C)], vbuf.at[slot],
                              vsem.at[slot]).start()

    def wait_chunk(slot):
        pltpu.make_async_copy(k_hbm.at[pl.ds(0, C)], kbuf.at[slot],
                              ksem.at[slot]).wait()
        pltpu.make_async_copy(v_hbm.at[pl.ds(0, C)], vbuf.at[slot],
                              vsem.at[slot]).wait()

    QB = qb_ref[...]
    m4 = jnp.full((1, BLH), NEG, jnp.float32)
    l4 = jnp.zeros((1, BLH), jnp.float32)
    o_list = [jnp.zeros((1, HD), jnp.float32) for _ in range(BL)]

    start_chunk(0, 0)
    for jc in range(NCH):
        slot = jc % 2
        if jc + 1 < NCH:
            start_chunk(jc + 1, 1 - slot)
        wait_chunk(slot)

        kmat = kbuf[slot].reshape(KEYS, HD)
        vmat = vbuf[slot].reshape(KEYS, HD)
        cnt = cntT_ref[pl.ds(jc * C, C), :]
        wcol = lax.dot_general(Rp, cnt, (((1,), (0,)), ((), ())),
                               preferred_element_type=jnp.float32)
        lnw = jnp.where(wcol > 0, jnp.log(wcol), -BIG)
        lnw32 = lax.dot_general(lnw, R4, (((1,), (0,)), ((), ())),
                                preferred_element_type=jnp.float32)
        s = lax.dot_general(kmat, QB, (((1,), (0,)), ((), ())),
                            preferred_element_type=jnp.float32) + lnw32
        m_new = jnp.maximum(m4, jnp.max(s, axis=0, keepdims=True))
        alpha = jnp.exp(m4 - m_new)
        p = jnp.exp(s - m_new)
        l4 = l4 * alpha + jnp.sum(p, axis=0, keepdims=True)
        m4 = m_new
        for i in range(BL):
            p_i = p[:, i * H:(i + 1) * H]
            pexp = lax.dot_general(p_i, R, (((1,), (0,)), ((), ())),
                                   preferred_element_type=jnp.float32)
            a_i = lax.dot_general(alpha[:, i * H:(i + 1) * H], R,
                                  (((1,), (0,)), ((), ())),
                                  preferred_element_type=jnp.float32)
            o_list[i] = o_list[i] * a_i + jnp.sum(pexp * vmat, axis=0,
                                                  keepdims=True)

    for i in range(BL):
        cm_send[pl.ds(i, 1), 0:HD] = o_list[i]
        cm_send[pl.ds(i, 1), HD:HD + H] = m4[:, i * H:(i + 1) * H]
        cm_send[pl.ds(i, 1), HD + H:HD + 2 * H] = l4[:, i * H:(i + 1) * H]

    rdma_x = pltpu.make_async_remote_copy(
        src_ref=cm_send, dst_ref=cm_recv,
        send_sem=sem_xs, recv_sem=sem_xr,
        device_id=xpeer, device_id_type=pl.DeviceIdType.MESH)
    rdma_x.start()
    rdma_x.wait()

    s_l = cm_send[...]
    s_r = cm_recv[...]
    o_l, m_l, l_l = s_l[:, :HD], s_l[:, HD:HD + H], s_l[:, HD + H:]
    o_r, m_r, l_r = s_r[:, :HD], s_r[:, HD:HD + H], s_r[:, HD + H:]
    m_t = jnp.maximum(m_l, m_r)
    a_l = jnp.exp(m_l - m_t)
    a_r = jnp.exp(m_r - m_t)
    l_t = l_l * a_l + l_r * a_r
    ex = lambda x: lax.dot_general(
        x, R, (((1,), (0,)), ((), ())),
        preferred_element_type=jnp.float32)
    yfin[...] = (o_l * ex(a_l) + o_r * ex(a_r)) / ex(l_t)

    rdma_y = pltpu.make_async_remote_copy(
        src_ref=yfin, dst_ref=yrecv,
        send_sem=sem_ys, recv_sem=sem_yr,
        device_id=ypeer, device_id_type=pl.DeviceIdType.MESH)
    rdma_y.start()
    rdma_y.wait()

    @pl.when(my_y == 0)
    def _():
        out_ref[0:BL, :] = yfin[...]
        out_ref[BL:B, :] = yrecv[...]

    @pl.when(my_y == 1)
    def _():
        out_ref[BL:B, :] = yfin[...]
        out_ref[0:BL, :] = yrecv[...]


def kernel(Q, K, V, bt, lens):
    my_x = lax.axis_index("x")
    my_y = lax.axis_index("y")

    b0 = my_y * BL
    Qh = lax.dynamic_slice_in_dim(Q, b0, BL, axis=0)
    bth = lax.dynamic_slice_in_dim(bt, b0, BL, axis=0)
    lensh = lax.dynamic_slice_in_dim(lens, b0, BL, axis=0)

    slot_idx = jnp.arange(NSLOTS)[None, :]
    valid = (slot_idx < lensh[:, None]) & ((bth // NPAGES) == my_x)
    p_off = bth - my_x * NPAGES
    eq = (p_off[:, :, None] == jnp.arange(NPAGES)[None, None, :])
    count = jnp.sum(eq & valid[:, :, None], axis=1)
    cntT = count.T.astype(jnp.float32)

    q = Qh[:, 0] * SCALE
    eye = jnp.eye(H, dtype=jnp.float32)
    QB = jnp.einsum('gh,hdi->gdih', eye,
                    q.transpose(1, 2, 0)).reshape(HD, BLH)

    K2 = K.reshape(NPAGES, BS, HD)
    V2 = V.reshape(NPAGES, BS, HD)

    out = pl.pallas_call(
        _body,
        out_shape=jax.ShapeDtypeStruct((B, HD), jnp.float32),
        in_specs=[
            pl.BlockSpec(memory_space=pltpu.VMEM),
            pl.BlockSpec(memory_space=pltpu.VMEM),
            pl.BlockSpec(memory_space=pl.ANY),
            pl.BlockSpec(memory_space=pl.ANY),
        ],
        out_specs=pl.BlockSpec(memory_space=pltpu.VMEM),
        scratch_shapes=[
            pltpu.VMEM((2, C, BS, HD), jnp.float32),
            pltpu.VMEM((2, C, BS, HD), jnp.float32),
            pltpu.VMEM((BL, HD + 2 * H), jnp.float32),
            pltpu.VMEM((BL, HD + 2 * H), jnp.float32),
            pltpu.VMEM((BL, HD), jnp.float32),
            pltpu.VMEM((BL, HD), jnp.float32),
            pltpu.SemaphoreType.DMA((2,)),
            pltpu.SemaphoreType.DMA((2,)),
            pltpu.SemaphoreType.DMA,
            pltpu.SemaphoreType.DMA,
            pltpu.SemaphoreType.DMA,
            pltpu.SemaphoreType.DMA,
        ],
        compiler_params=pltpu.CompilerParams(
            collective_id=0, vmem_limit_bytes=100 * 1024 * 1024),
    )(cntT, QB, K2, V2)
    return out.reshape(B, 1, H, D)
